# baseline (speedup 1.0000x reference)
import functools

import jax
import jax.numpy as jnp
from jax import lax
from jax.experimental import pallas as pl
from jax.experimental.pallas import tpu as pltpu
from jax.experimental.pallas import tpu_sc as plsc

N_CAT = 26
CARD = 100000
N_NUM = 13
D_H = 16
D_M = 64

NC, NS = 2, 16
NW = NC * NS
LANES = 16


def _sc_gather_body(nb, n_chunks, batch, xcat_hbm, table_hbm, out_hbm,
                    xcat_v, idx_v, rows_v, sem):
    wid = lax.axis_index("s") * NC + lax.axis_index("c")
    iota = lax.iota(jnp.int32, LANES)
    for c in range(n_chunks):
        b0 = wid * (nb * n_chunks) + c * nb
        pltpu.sync_copy(xcat_hbm.at[pl.ds(b0 * N_CAT, nb * N_CAT)], xcat_v)
        for j in range(N_CAT):
            off = jnp.int32(j * CARD)

            def body(i, _, j=j, off=off):
                src = (i * LANES + iota) * N_CAT + j
                vals = plsc.load_gather(xcat_v, [src])
                idx_v[pl.ds(j * nb + i * LANES, LANES)] = vals + off
                return 0

            lax.fori_loop(0, nb // LANES, body, 0, unroll=4)
        pltpu.async_copy(table_hbm.at[idx_v], rows_v, sem).wait()
        for j in range(N_CAT):
            pltpu.sync_copy(rows_v.at[pl.ds(j * nb, nb)],
                            out_hbm.at[pl.ds(j * batch + b0, nb)])


def _sc_gather(x_cat, emb_table):
    batch = x_cat.shape[0]
    per_w = batch // NW
    nb = min(256, per_w)
    n_chunks = per_w // nb
    mesh = plsc.VectorSubcoreMesh(core_axis_name="c", subcore_axis_name="s",
                                  num_cores=NC, num_subcores=NS)
    body = functools.partial(_sc_gather_body, nb, n_chunks, batch)
    run = pl.kernel(
        body,
        out_type=jax.ShapeDtypeStruct((N_CAT * batch, D_H), jnp.float32),
        mesh=mesh,
        scratch_types=[
            pltpu.VMEM((nb * N_CAT,), jnp.int32),
            pltpu.VMEM((nb * N_CAT,), jnp.int32),
            pltpu.VMEM((nb * N_CAT, D_H), jnp.float32),
            pltpu.SemaphoreType.DMA,
        ],
        compiler_params=pltpu.CompilerParams(needs_layout_passes=False,
                                             use_tc_tiling_on_sc=False),
    )
    return run(x_cat.reshape(-1), emb_table)


def _tc_body(xnum_ref, gath_ref, nw_ref, nbias_ref, nproj_ref, cb_ref, cp_ref,
             out_ref):
    bb = xnum_ref.shape[0]
    w2 = jnp.dot(nw_ref[:], nproj_ref[:], preferred_element_type=jnp.float32)
    b2 = jnp.dot(nbias_ref[:], nproj_ref[:], preferred_element_type=jnp.float32)
    bc2 = jnp.dot(cb_ref[:], cp_ref[:], preferred_element_type=jnp.float32)

    w2t = jnp.concatenate([w2] * N_NUM, axis=1)
    b2t = jnp.concatenate([b2] * N_NUM, axis=1)
    row = lax.broadcasted_iota(jnp.int32, (N_NUM, N_NUM * D_M), 0)
    col = lax.broadcasted_iota(jnp.int32, (N_NUM, N_NUM * D_M), 1) // D_M
    blockdiag = row == col
    wn = jnp.where(blockdiag, w2t, 0.0)
    bn = jnp.sum(jnp.where(blockdiag, b2t, 0.0), axis=0, keepdims=True)
    out_n = jnp.dot(xnum_ref[:], wn, preferred_element_type=jnp.float32) + bn

    parts = [out_n]
    cp = cp_ref[:]
    for j in range(N_CAT):
        g = gath_ref[j]
        yj = jnp.dot(g, cp, preferred_element_type=jnp.float32)
        bj = lax.broadcast_in_dim(bc2[j], (bb, D_M), (1,))
        parts.append(yj + bj)
    out_ref[:] = jnp.concatenate(parts, axis=1)


def _tc_fused(x_num, gath, num_weight, num_bias, num_proj, cat_bias, cat_proj):
    batch = x_num.shape[0]
    bb = 512
    grid = (batch // bb,)
    d_out = (N_NUM + N_CAT) * D_M
    return pl.pallas_call(
        _tc_body,
        grid=grid,
        in_specs=[
            pl.BlockSpec((bb, N_NUM), lambda i: (i, 0)),
            pl.BlockSpec((N_CAT, bb, D_H), lambda i: (0, i, 0)),
            pl.BlockSpec((N_NUM, D_H), lambda i: (0, 0)),
            pl.BlockSpec((N_NUM, D_H), lambda i: (0, 0)),
            pl.BlockSpec((D_H, D_M), lambda i: (0, 0)),
            pl.BlockSpec((N_CAT, D_H), lambda i: (0, 0)),
            pl.BlockSpec((D_H, D_M), lambda i: (0, 0)),
        ],
        out_specs=pl.BlockSpec((bb, d_out), lambda i: (i, 0)),
        out_shape=jax.ShapeDtypeStruct((batch, d_out), jnp.float32),
    )(x_num, gath, num_weight, num_bias, num_proj, cat_bias, cat_proj)


def kernel(x_num, x_cat, num_weight, num_bias, num_proj, emb_table, cat_bias,
           cat_proj):
    batch = x_cat.shape[0]
    gath = _sc_gather(x_cat, emb_table).reshape(N_CAT, batch, D_H)
    out = _tc_fused(x_num, gath, num_weight, num_bias, num_proj, cat_bias,
                    cat_proj)
    return out.reshape(batch, N_NUM + N_CAT, D_M)
